# baseline (speedup 1.0000x reference)
import functools
import math

import jax
import jax.numpy as jnp
from jax import lax
from jax.experimental import pallas as pl
from jax.experimental.pallas import tpu as pltpu

_BETA = 0.6
_ALPHA = 0.25
_EPS = 1e-5
_E = math.e
_VMEM_LIMIT = 48 * 1024 * 1024


def _prelu(x):
    return jnp.where(x >= 0.0, x, _ALPHA * x)


def _bf16(x):
    return x.astype(jnp.bfloat16)


def _encoder_kernel(adj_ref, feat_ref, wg_ref, bg_ref, wm_ref, vm_ref,
                    pred_ref, tproj_ref, *, d):
    adj = _bf16(adj_ref[0])
    feat = _bf16(feat_ref[0])

    xw = jnp.dot(feat, wg_ref[...], preferred_element_type=jnp.float32)
    g = jnp.dot(adj, _bf16(xw),
                preferred_element_type=jnp.float32) + bg_ref[...]
    o_rep = _prelu(g[:, :d])
    t_rep = _prelu(g[:, d:])

    def mlp(x, wi, vi):
        y = jnp.dot(_bf16(x), wm_ref[wi],
                    preferred_element_type=jnp.float32) + vm_ref[vi]
        mu = jnp.mean(y, axis=0, keepdims=True)
        var = jnp.mean(jnp.square(y - mu), axis=0, keepdims=True)
        yh = (y - mu) * lax.rsqrt(var + _EPS) * vm_ref[vi + 1] + vm_ref[vi + 2]
        z = _prelu(yh)
        return jnp.dot(_bf16(z), wm_ref[wi + 1],
                       preferred_element_type=jnp.float32) + vm_ref[vi + 3]

    def unit(v):
        ss = jnp.sum(v * v, axis=-1, keepdims=True)
        return v * lax.rsqrt(jnp.maximum(ss, 1e-24))

    o_proj = mlp(o_rep, 0, 0)
    o_pred = mlp(o_proj, 2, 4)
    t_proj = mlp(t_rep, 4, 8)

    pred_ref[0] = _bf16(unit(o_pred))
    tproj_ref[0] = _bf16(unit(t_proj))


def _run_encoder(adj, feat, wg, bg, wm, vm, d):
    n = adj.shape[1]
    f = feat.shape[-1]
    body = functools.partial(_encoder_kernel, d=d)
    in_specs = [
        pl.BlockSpec((1, n, n), lambda v: (v, 0, 0)),
        pl.BlockSpec((1, n, f), lambda v: (v, 0, 0)),
        pl.BlockSpec(wg.shape, lambda v: (0, 0)),
        pl.BlockSpec(bg.shape, lambda v: (0, 0)),
        pl.BlockSpec(wm.shape, lambda v: (0, 0, 0)),
        pl.BlockSpec(vm.shape, lambda v: (0, 0, 0)),
    ]
    out_specs = (pl.BlockSpec((1, n, d), lambda v: (v, 0, 0)),
                 pl.BlockSpec((1, n, d), lambda v: (v, 0, 0)))
    out_shape = (jax.ShapeDtypeStruct((2, n, d), jnp.bfloat16),
                 jax.ShapeDtypeStruct((2, n, d), jnp.bfloat16))
    return pl.pallas_call(
        body,
        grid=(2,),
        in_specs=in_specs,
        out_specs=out_specs,
        out_shape=out_shape,
        compiler_params=pltpu.CompilerParams(
            dimension_semantics=("parallel",),
            vmem_limit_bytes=_VMEM_LIMIT),
    )(adj, feat, wg, bg, wm, vm)


def _loss_kernel(pr_ref, tr_ref, big1_ref, big2_ref, o_ref, *, n):
    h1b = pr_ref[0]
    h2b = pr_ref[1]
    z1b = _f32(tr_ref[0])
    z2b = _f32(tr_ref[1])

    def expdot(a, c):
        s = lax.dot_general(a, c, (((1,), (1,)), ((), ())),
                            preferred_element_type=jnp.float32)
        return jnp.exp(s)

    e1 = expdot(h1b, big1_ref[...])
    e2 = expdot(h2b, big2_ref[...])

    def seg(e, lo, hi):
        return jnp.sum(e[:, lo:hi], axis=-1, keepdims=True)

    def col(v):
        return jnp.sum(v, axis=0, keepdims=True)

    net1 = col(jnp.log(seg(e1, 0, 2 * n) - _E))
    net2 = col(jnp.log(seg(e2, 0, 2 * n) - _E))
    view1 = col(jnp.log(seg(e1, 2 * n, 3 * n)))
    view2 = col(jnp.log(seg(e2, 2 * n, 3 * n)))

    h1f = _f32(h1b)
    h2f = _f32(h2b)
    d12 = col(jnp.sum(h1f * h2f, axis=-1, keepdims=True))
    d1z2 = col(jnp.sum(h1f * z2b, axis=-1, keepdims=True))
    d2z1 = col(jnp.sum(h2f * z1b, axis=-1, keepdims=True))

    part = (_BETA * (net1 + net2 - 2.0 * d12)
            + (1.0 - _BETA) * (view1 + view2 - d1z2 - d2z1))
    o_ref[...] = jnp.broadcast_to(part * (0.5 / n), o_ref.shape)


def _f32(x):
    return x.astype(jnp.float32)


def _run_loss(pred, tproj):
    _, n, d = pred.shape
    r = 256 if n % 256 == 0 else n
    nb = n // r
    h1, h2 = pred[0], pred[1]
    z1, z2 = tproj[0], tproj[1]
    big1 = jnp.concatenate([h1, h2, z2], axis=0)
    big2 = jnp.concatenate([h2, h1, z1], axis=0)
    body = functools.partial(_loss_kernel, n=n)
    out = pl.pallas_call(
        body,
        grid=(nb,),
        in_specs=[
            pl.BlockSpec((2, r, d), lambda b: (0, b, 0)),
            pl.BlockSpec((2, r, d), lambda b: (0, b, 0)),
            pl.BlockSpec((3 * n, d), lambda b: (0, 0)),
            pl.BlockSpec((3 * n, d), lambda b: (0, 0)),
        ],
        out_specs=pl.BlockSpec((1, 1, 128), lambda b: (b, 0, 0)),
        out_shape=jax.ShapeDtypeStruct((nb, 1, 128), jnp.float32),
        compiler_params=pltpu.CompilerParams(
            dimension_semantics=("parallel",),
            vmem_limit_bytes=_VMEM_LIMIT),
    )(pred, tproj, big1, big2)
    return jnp.sum(out[:, 0, 0])


def kernel(adj, feat,
           online_gcn_w, online_gcn_b,
           online_proj_w1, online_proj_b1, online_proj_gamma,
           online_proj_beta, online_proj_w2, online_proj_b2,
           target_gcn_w, target_gcn_b,
           target_proj_w1, target_proj_b1, target_proj_gamma,
           target_proj_beta, target_proj_w2, target_proj_b2,
           pred_w1, pred_b1, pred_gamma, pred_beta, pred_w2, pred_b2):
    d = online_gcn_w.shape[1]

    wg = _bf16(jnp.concatenate([online_gcn_w, target_gcn_w], axis=1))
    bg = jnp.concatenate([online_gcn_b, target_gcn_b], axis=1)
    wm = _bf16(jnp.stack([online_proj_w1, online_proj_w2,
                          pred_w1, pred_w2,
                          target_proj_w1, target_proj_w2]))
    vm = jnp.stack([
        online_proj_b1, online_proj_gamma, online_proj_beta, online_proj_b2,
        pred_b1, pred_gamma, pred_beta, pred_b2,
        target_proj_b1, target_proj_gamma, target_proj_beta, target_proj_b2,
    ])

    pred, tproj = _run_encoder(adj, feat, wg, bg, wm, vm, d)
    return _run_loss(pred, tproj)
